# Initial kernel scaffold; baseline (speedup 1.0000x reference)
import jax
import jax.numpy as jnp
from jax.experimental import pallas as pl


def kernel(x_text, x_vis, text_enc_params, text_dec_params, vis_enc_params, vis_dec_params, lora_A, text_B, vis_B):
    raise NotImplementedError("write your pallas kernel here")


# speedup vs baseline: 1.8179x; 1.8179x over previous
import jax
import jax.numpy as jnp
from jax.experimental import pallas as pl
from jax.experimental.pallas import tpu as pltpu

BATCH = 4096
EDIM = 64
NE = 256
NLEV = 3
TILE = 512

_F32 = jnp.float32


def _dot(a, b):
    return jax.lax.dot_general(a, b, (((1,), (0,)), ((), ())),
                               preferred_element_type=_F32)


def _dot_t(a, b):
    return jax.lax.dot_general(a, b, (((1,), (1,)), ((), ())),
                               preferred_element_type=_F32)


def _round_bf16(h):
    u = jax.lax.bitcast_convert_type(h, jnp.int32)
    bias = 0x7FFF + ((u >> 16) & 1)
    u = (u + bias) & jnp.int32(-65536)
    return jax.lax.bitcast_convert_type(u, _F32)


def _mlp(h, wbs):
    n = len(wbs)
    for i in range(n):
        W, b = wbs[i]
        h = _dot(h, W[...]) + b[...]
        if i < n - 1:
            h = _round_bf16(jnp.maximum(h, 0.0))
    return h


def _rq(z, cbs):
    r = z
    xq = jnp.zeros_like(z)
    loss = jnp.zeros((), dtype=_F32)
    idxs = []
    for cb in cbs:
        d = (jnp.sum(r * r, axis=1, keepdims=True)
             - 2.0 * _dot_t(r, cb)
             + jnp.sum(cb * cb, axis=1)[None, :])
        idx = jnp.argmin(d, axis=1).astype(jnp.int32)
        onehot = (jax.lax.broadcasted_iota(jnp.int32, d.shape, 1)
                  == idx[:, None]).astype(_F32)
        sel = jax.lax.dot_general(onehot, cb, (((1,), (0,)), ((), ())),
                                  preferred_element_type=_F32,
                                  precision=jax.lax.Precision.HIGHEST)
        diff = sel - r
        loss = loss + jnp.sum(diff * diff)
        xq = xq + sel
        r = r - sel
        idxs.append(idx)
    return xq, loss, jnp.stack(idxs, axis=1)


def _body(xt_ref, xv_ref, *refs):
    wrefs = refs[:41]
    out_t_ref, out_v_ref, idx_t_ref, idx_v_ref, lt_ref, lv_ref = refs[41:]

    pos = [0]

    def take_pairs(n):
        out = []
        for _ in range(n):
            out.append((wrefs[pos[0]], wrefs[pos[0] + 1]))
            pos[0] += 2
        return out

    def take(n):
        out = list(wrefs[pos[0]:pos[0] + n])
        pos[0] += n
        return out

    te = take_pairs(4)
    td = take_pairs(4)
    ve = take_pairs(4)
    vd = take_pairs(4)
    A = take(3)
    tB = take(3)
    vB = take(3)

    cbs_t = [_dot(A[l][...], tB[l][...]) for l in range(NLEV)]
    cbs_v = [_dot(A[l][...], vB[l][...]) for l in range(NLEV)]

    z_t = _mlp(xt_ref[...], te)
    z_v = _mlp(xv_ref[...], ve)

    xq_t, loss_t, idx_t = _rq(z_t, cbs_t)
    xq_v, loss_v, idx_v = _rq(z_v, cbs_v)

    out_t_ref[...] = _mlp(xq_t, td)
    out_v_ref[...] = _mlp(xq_v, vd)
    idx_t_ref[...] = idx_t
    idx_v_ref[...] = idx_v

    i = pl.program_id(0)

    @pl.when(i == 0)
    def _init():
        lt_ref[...] = jnp.zeros((1, 1), _F32)
        lv_ref[...] = jnp.zeros((1, 1), _F32)

    lt_ref[...] += jnp.reshape(loss_t, (1, 1))
    lv_ref[...] += jnp.reshape(loss_v, (1, 1))


def kernel(x_text, x_vis, text_enc_params, text_dec_params, vis_enc_params,
           vis_dec_params, lora_A, text_B, vis_B):
    flat = []
    for params in (text_enc_params, text_dec_params, vis_enc_params,
                   vis_dec_params):
        for W, b in params:
            flat.append(W)
            flat.append(b.reshape(1, -1))
    flat.extend(lora_A)
    flat.extend(text_B)
    flat.extend(vis_B)

    grid = (BATCH // TILE,)

    def row_spec(cols):
        return pl.BlockSpec((TILE, cols), lambda i: (i, 0))

    def full_spec(shape):
        return pl.BlockSpec(shape, lambda i: (0,) * len(shape))

    in_specs = [row_spec(x_text.shape[1]), row_spec(x_vis.shape[1])]
    in_specs += [full_spec(a.shape) for a in flat]

    out_shape = [
        jax.ShapeDtypeStruct((BATCH, x_text.shape[1]), _F32),
        jax.ShapeDtypeStruct((BATCH, x_vis.shape[1]), _F32),
        jax.ShapeDtypeStruct((BATCH, NLEV), jnp.int32),
        jax.ShapeDtypeStruct((BATCH, NLEV), jnp.int32),
        jax.ShapeDtypeStruct((1, 1), _F32),
        jax.ShapeDtypeStruct((1, 1), _F32),
    ]
    out_specs = [
        row_spec(x_text.shape[1]),
        row_spec(x_vis.shape[1]),
        pl.BlockSpec((TILE, NLEV), lambda i: (i, 0)),
        pl.BlockSpec((TILE, NLEV), lambda i: (i, 0)),
        full_spec((1, 1)),
        full_spec((1, 1)),
    ]

    out_t, out_v, idx_t, idx_v, lt, lv = pl.pallas_call(
        _body,
        grid=grid,
        in_specs=in_specs,
        out_specs=out_specs,
        out_shape=out_shape,
    )(x_text, x_vis, *flat)

    scale = 1.25 / (BATCH * EDIM)
    return (out_t, out_v, lt[0, 0] * scale, lv[0, 0] * scale, idx_t, idx_v)
